# baseline (speedup 1.0000x reference)
import jax
import jax.numpy as jnp
import numpy as np
from jax.experimental import pallas as pl

_T = 512
_DH = 64
_STRIDE = 7
_LOCAL = 4


def _mask_scaled():
    t = np.arange(_T)
    dlt = np.abs(t[:, None] - t[None, :])
    m = (dlt <= _LOCAL // 2) | (dlt % (_STRIDE + 1) == 0)
    return (m.astype(np.float32) / np.sqrt(_DH)).astype(np.float32)


def _one_head_t(qt, kt, vt, msk):
    s = jax.lax.dot_general(
        qt, kt, dimension_numbers=(((0,), (0,)), ((), ())),
        preferred_element_type=jnp.float32)
    ee = jnp.exp(s * msk) - 1.0
    va = jnp.concatenate([vt, jnp.ones((1, _T), jnp.float32)], axis=0)
    na = jax.lax.dot_general(
        va, ee, dimension_numbers=(((1,), (1,)), ((), ())),
        preferred_element_type=jnp.float32)
    vsum = jnp.sum(vt, axis=1, keepdims=True)
    return (na[:_DH] + vsum) / (na[_DH:] + np.float32(_T))


def _pair_kernel(m_ref, q_ref, k_ref, v_ref, o_ref):
    msk = m_ref[...]
    o0 = _one_head_t(q_ref[0, 0, 0], k_ref[0, 0, 0], v_ref[0, 0, 0], msk)
    o1 = _one_head_t(q_ref[0, 1, 0], k_ref[0, 1, 0], v_ref[0, 0, 1], msk)
    o_ref[0, :, :] = jnp.concatenate(
        [o0.transpose(1, 0), o1.transpose(1, 0)], axis=1)


@jax.jit
def _run(q, k, v):
    B, N, H, T, Dh = q.shape
    J = H // 2
    qt = jnp.swapaxes(q, 3, 4)
    kt = jnp.swapaxes(k, 3, 4)
    vt = jnp.swapaxes(v, 3, 4)
    msk = jnp.asarray(_mask_scaled())
    m_spec = pl.BlockSpec((T, T), lambda b, n, j: (0, 0))
    qk_spec = pl.BlockSpec((1, 2, 1, Dh, T), lambda b, n, j: (b, j, n, 0, 0))
    v_spec = pl.BlockSpec((1, 1, 2, Dh, T), lambda b, n, j: (b, n, j, 0, 0))
    out_spec = pl.BlockSpec((1, T, 2 * Dh), lambda b, n, j: (b, 0, n * J + j))
    out = pl.pallas_call(
        _pair_kernel,
        grid=(B, N, J),
        in_specs=[m_spec, qk_spec, qk_spec, v_spec],
        out_specs=out_spec,
        out_shape=jax.ShapeDtypeStruct((B, T, N * H * Dh), jnp.float32),
    )(msk, qt, kt, vt)
    return out.reshape(B, T, N, H * Dh)


def kernel(q, k, v, dims):
    return _run(q, k, v)


# speedup vs baseline: 1.2895x; 1.2895x over previous
import jax
import jax.numpy as jnp
import numpy as np
from jax.experimental import pallas as pl

_T = 512
_DH = 64
_STRIDE = 7
_LOCAL = 4


def _mask_scaled():
    t = np.arange(_T)
    dlt = np.abs(t[:, None] - t[None, :])
    m = (dlt <= _LOCAL // 2) | (dlt % (_STRIDE + 1) == 0)
    return (m.astype(np.float32) / np.sqrt(_DH)).astype(np.float32)


def _one_head_t(qt, kt, vt, msk):
    s = jax.lax.dot_general(
        qt, kt, dimension_numbers=(((0,), (0,)), ((), ())),
        preferred_element_type=jnp.float32)
    ee = jnp.exp(s * msk) - 1.0
    va = jnp.concatenate([vt, jnp.ones((1, _T), jnp.float32)], axis=0)
    na = jax.lax.dot_general(
        va, ee, dimension_numbers=(((1,), (1,)), ((), ())),
        preferred_element_type=jnp.float32)
    vsum = jnp.sum(vt, axis=1, keepdims=True)
    return (na[:_DH] + vsum) / (na[_DH:] + np.float32(_T))


def _quad_kernel(m_ref, q_ref, k_ref, v_ref, o_ref):
    msk = m_ref[...]
    outs = [
        _one_head_t(q_ref[0, i, 0], k_ref[0, i, 0], v_ref[0, 0, i], msk)
        for i in range(4)
    ]
    o_ref[0, :, :] = jnp.concatenate(outs, axis=0)


@jax.jit
def _run(q, k, v):
    B, N, H, T, Dh = q.shape
    J = H // 4
    qt = jnp.swapaxes(q, 3, 4)
    kt = jnp.swapaxes(k, 3, 4)
    vt = jnp.swapaxes(v, 3, 4)
    msk = jnp.asarray(_mask_scaled())
    m_spec = pl.BlockSpec((T, T), lambda b, n, j: (0, 0))
    qk_spec = pl.BlockSpec((1, 4, 1, Dh, T), lambda b, n, j: (b, j, n, 0, 0))
    v_spec = pl.BlockSpec((1, 1, 4, Dh, T), lambda b, n, j: (b, n, j, 0, 0))
    out_spec = pl.BlockSpec((1, 4 * Dh, T), lambda b, n, j: (b, n * J + j, 0))
    out = pl.pallas_call(
        _quad_kernel,
        grid=(B, N, J),
        in_specs=[m_spec, qk_spec, qk_spec, v_spec],
        out_specs=out_spec,
        out_shape=jax.ShapeDtypeStruct((B, N * H * Dh, T), jnp.float32),
    )(msk, qt, kt, vt)
    return jnp.swapaxes(out, 1, 2).reshape(B, T, N, H * Dh)


def kernel(q, k, v, dims):
    return _run(q, k, v)
